# Initial kernel scaffold; baseline (speedup 1.0000x reference)
import jax
import jax.numpy as jnp
from jax.experimental import pallas as pl


def kernel(x, edge_index, W, b):
    raise NotImplementedError("write your pallas kernel here")


# speedup vs baseline: 9.5926x; 9.5926x over previous
import functools

import jax
import jax.numpy as jnp
from jax import lax
from jax.experimental import pallas as pl
from jax.experimental.pallas import tpu as pltpu
from jax.experimental.pallas import tpu_sc as plsc

_N = 10000
_E = 320000
_DIN = 128
_DOUT = 256

_NC = 2
_NS = 16
_NW = _NC * _NS
_EPW = _E // _NW
_CHUNK = 80
_NCHUNK = _EPW // _CHUNK
_RPAD = 10240
_RPT = _RPAD // _NS
_ZC = 128
_MBLK = 512

_mesh = plsc.VectorSubcoreMesh(core_axis_name="c", subcore_axis_name="s")


@functools.partial(
    pl.kernel,
    mesh=_mesh,
    out_type=jax.ShapeDtypeStruct((_NC, _RPAD, _DIN), jnp.float32),
    scratch_types=[
        pltpu.VMEM((_CHUNK, _DIN), jnp.float32),
        pltpu.VMEM((_NCHUNK, _CHUNK), jnp.int32),
        pltpu.VMEM((_NCHUNK, _CHUNK), jnp.int32),
        pltpu.VMEM_SHARED((_RPAD, _DIN), jnp.float32),
        pltpu.SemaphoreType.DMA,
    ],
)
def _sc_segsum(src_hbm, dst_hbm, x_hbm, zeros_hbm, out_hbm,
               rows_v, src_v, dst_v, acc_sh, sem):
    c = lax.axis_index("c")
    s = lax.axis_index("s")
    wid = c * _NS + s
    pltpu.sync_copy(src_hbm.at[wid], src_v)
    pltpu.sync_copy(dst_hbm.at[wid], dst_v)
    for k in range(_RPT // _ZC):
        pltpu.sync_copy(zeros_hbm, acc_sh.at[pl.ds(s * _RPT + k * _ZC, _ZC)])
    plsc.subcore_barrier()

    def body(j, carry):
        pltpu.async_copy(x_hbm.at[src_v.at[j]], rows_v, sem).wait()
        pltpu.sync_copy(rows_v, acc_sh.at[dst_v.at[j]], add=True)
        return carry

    lax.fori_loop(0, _NCHUNK, body, 0)
    plsc.subcore_barrier()
    for k in range(_RPT // _ZC):
        r0 = s * _RPT + k * _ZC
        pltpu.sync_copy(acc_sh.at[pl.ds(r0, _ZC)], out_hbm.at[c, pl.ds(r0, _ZC)])


def _tc_body(a_ref, w_ref, b_ref, o_ref):
    blk = a_ref[0] + a_ref[1]
    y = jnp.dot(blk, w_ref[...], preferred_element_type=jnp.float32)
    o_ref[...] = jnp.maximum(y + b_ref[...], 0.0)


_tc_matmul = pl.pallas_call(
    _tc_body,
    grid=(_RPAD // _MBLK,),
    in_specs=[
        pl.BlockSpec((_NC, _MBLK, _DIN), lambda i: (0, i, 0)),
        pl.BlockSpec((_DIN, _DOUT), lambda i: (0, 0)),
        pl.BlockSpec((1, _DOUT), lambda i: (0, 0)),
    ],
    out_specs=pl.BlockSpec((_MBLK, _DOUT), lambda i: (i, 0)),
    out_shape=jax.ShapeDtypeStruct((_RPAD, _DOUT), jnp.float32),
)


def kernel(x, edge_index, W, b):
    ei = edge_index.astype(jnp.int32)
    src = ei[0].reshape(_NW, _NCHUNK, _CHUNK)
    dst = ei[1].reshape(_NW, _NCHUNK, _CHUNK)
    zeros = jnp.zeros((_ZC, _DIN), jnp.float32)
    acc = _sc_segsum(src, dst, x, zeros)
    out = _tc_matmul(acc, W, b.reshape(1, _DOUT))
    return out[:_N]
